# initial kernel scaffold (unmeasured)
import jax
import jax.numpy as jnp
from jax import lax
from jax.experimental import pallas as pl
from jax.experimental.pallas import tpu as pltpu


def kernel(
    x,
):
    def body(*refs):
        pass

    out_shape = jax.ShapeDtypeStruct(..., jnp.float32)
    return pl.pallas_call(body, out_shape=out_shape)(...)


# baseline (device time: 45432 ns/iter reference)
import jax
import jax.numpy as jnp
from jax import lax
from jax.experimental import pallas as pl
from jax.experimental.pallas import tpu as pltpu

N_DEV = 4


def kernel(x):
    _, m, n_total = x.shape
    n_per = n_total // N_DEV

    def body(x_ref, out_ref, sbuf, rbuf, send_sems, recv_sems):
        my = lax.axis_index("i")
        left = (my + N_DEV - 1) % N_DEV
        right = (my + 1) % N_DEV

        barrier_sem = pltpu.get_barrier_semaphore()
        for nbr in (left, right):
            pl.semaphore_signal(
                barrier_sem, inc=1,
                device_id=(nbr,), device_id_type=pl.DeviceIdType.MESH,
            )
        pl.semaphore_wait(barrier_sem, 2)

        def local_chunk(c):
            return x_ref[0, :, pl.ds(c * n_per, n_per)]

        sbuf[0] = local_chunk((my + N_DEV - 1) % N_DEV)

        for s in range(N_DEV - 1):
            rdma = pltpu.make_async_remote_copy(
                src_ref=sbuf.at[s],
                dst_ref=rbuf.at[s],
                send_sem=send_sems.at[s],
                recv_sem=recv_sems.at[s],
                device_id=(right,),
                device_id_type=pl.DeviceIdType.MESH,
            )
            rdma.start()
            rdma.wait()

            c = (my + 2 * N_DEV - s - 2) % N_DEV
            if s < N_DEV - 2:
                sbuf[s + 1] = rbuf[s] + local_chunk(c)
            else:
                out_ref[:, :] = rbuf[s] + local_chunk(c)

    return pl.pallas_call(
        body,
        out_shape=jax.ShapeDtypeStruct((m, n_per), jnp.float32),
        in_specs=[pl.BlockSpec(memory_space=pltpu.VMEM)],
        out_specs=pl.BlockSpec(memory_space=pltpu.VMEM),
        scratch_shapes=[
            pltpu.VMEM((N_DEV - 1, m, n_per), jnp.float32),
            pltpu.VMEM((N_DEV - 1, m, n_per), jnp.float32),
            pltpu.SemaphoreType.DMA((N_DEV - 1,)),
            pltpu.SemaphoreType.DMA((N_DEV - 1,)),
        ],
        compiler_params=pltpu.CompilerParams(collective_id=0),
    )(x)


# device time: 28715 ns/iter; 1.5822x vs baseline; 1.5822x over previous
import jax
import jax.numpy as jnp
from jax import lax
from jax.experimental import pallas as pl
from jax.experimental.pallas import tpu as pltpu

N_DEV = 4


def kernel(x):
    _, m, n_total = x.shape
    n_per = n_total // N_DEV
    h = n_per // 2

    def body(x_ref, out_ref, sb_cw, rb_cw, sb_ccw, rb_ccw,
             ss_cw, rs_cw, ss_ccw, rs_ccw):
        my = lax.axis_index("i")
        left = (my + N_DEV - 1) % N_DEV
        right = (my + 1) % N_DEV

        barrier_sem = pltpu.get_barrier_semaphore()
        for nbr in (left, right):
            pl.semaphore_signal(
                barrier_sem, inc=1,
                device_id=(nbr,), device_id_type=pl.DeviceIdType.MESH,
            )
        pl.semaphore_wait(barrier_sem, 2)

        def cw_half(c):
            return x_ref[0, :, pl.ds(c * n_per, h)]

        def ccw_half(c):
            return x_ref[0, :, pl.ds(c * n_per + h, h)]

        sb_cw[0] = cw_half((my + N_DEV - 1) % N_DEV)
        sb_ccw[0] = ccw_half((my + 1) % N_DEV)

        for s in range(N_DEV - 1):
            rd_cw = pltpu.make_async_remote_copy(
                src_ref=sb_cw.at[s], dst_ref=rb_cw.at[s],
                send_sem=ss_cw.at[s], recv_sem=rs_cw.at[s],
                device_id=(right,), device_id_type=pl.DeviceIdType.MESH,
            )
            rd_ccw = pltpu.make_async_remote_copy(
                src_ref=sb_ccw.at[s], dst_ref=rb_ccw.at[s],
                send_sem=ss_ccw.at[s], recv_sem=rs_ccw.at[s],
                device_id=(left,), device_id_type=pl.DeviceIdType.MESH,
            )
            rd_cw.start()
            rd_ccw.start()
            rd_cw.wait()
            rd_ccw.wait()

            c_cw = (my + 2 * N_DEV - s - 2) % N_DEV
            c_ccw = (my + s + 2) % N_DEV
            if s < N_DEV - 2:
                sb_cw[s + 1] = rb_cw[s] + cw_half(c_cw)
                sb_ccw[s + 1] = rb_ccw[s] + ccw_half(c_ccw)
            else:
                out_ref[:, 0:h] = rb_cw[s] + cw_half(c_cw)
                out_ref[:, h:n_per] = rb_ccw[s] + ccw_half(c_ccw)

    return pl.pallas_call(
        body,
        out_shape=jax.ShapeDtypeStruct((m, n_per), jnp.float32),
        in_specs=[pl.BlockSpec(memory_space=pltpu.VMEM)],
        out_specs=pl.BlockSpec(memory_space=pltpu.VMEM),
        scratch_shapes=[
            pltpu.VMEM((N_DEV - 1, m, h), jnp.float32),
            pltpu.VMEM((N_DEV - 1, m, h), jnp.float32),
            pltpu.VMEM((N_DEV - 1, m, h), jnp.float32),
            pltpu.VMEM((N_DEV - 1, m, h), jnp.float32),
            pltpu.SemaphoreType.DMA((N_DEV - 1,)),
            pltpu.SemaphoreType.DMA((N_DEV - 1,)),
            pltpu.SemaphoreType.DMA((N_DEV - 1,)),
            pltpu.SemaphoreType.DMA((N_DEV - 1,)),
        ],
        compiler_params=pltpu.CompilerParams(collective_id=0),
    )(x)


# device time: 25324 ns/iter; 1.7940x vs baseline; 1.1339x over previous
import jax
import jax.numpy as jnp
from jax import lax
from jax.experimental import pallas as pl
from jax.experimental.pallas import tpu as pltpu

N_DEV = 4
SEG = 4


def kernel(x):
    _, m, n_total = x.shape
    n_per = n_total // N_DEV
    h = n_per // 2
    sm = m // SEG

    def body(x_ref, out_ref, sb_cw, rb_cw, sb_ccw, rb_ccw,
             ss_cw, rs_cw, ss_ccw, rs_ccw):
        my = lax.axis_index("i")
        left = (my + N_DEV - 1) % N_DEV
        right = (my + 1) % N_DEV

        barrier_sem = pltpu.get_barrier_semaphore()
        for nbr in (left, right):
            pl.semaphore_signal(
                barrier_sem, inc=1,
                device_id=(nbr,), device_id_type=pl.DeviceIdType.MESH,
            )
        pl.semaphore_wait(barrier_sem, 2)

        def x_seg(c, j, half):
            return x_ref[0, pl.ds(j * sm, sm), pl.ds(c * n_per + half * h, h)]

        def make(sb, rb, ss, rs, s, j, dev):
            return pltpu.make_async_remote_copy(
                src_ref=sb.at[s, pl.ds(j * sm, sm), :],
                dst_ref=rb.at[s, pl.ds(j * sm, sm), :],
                send_sem=ss.at[s, j], recv_sem=rs.at[s, j],
                device_id=(dev,), device_id_type=pl.DeviceIdType.MESH,
            )

        rdmas = {}

        c0_cw = (my + N_DEV - 1) % N_DEV
        c0_ccw = (my + 1) % N_DEV
        for j in range(SEG):
            sb_cw[0, pl.ds(j * sm, sm), :] = x_seg(c0_cw, j, 0)
            rd = make(sb_cw, rb_cw, ss_cw, rs_cw, 0, j, right)
            rd.start()
            rdmas[("cw", 0, j)] = rd
            sb_ccw[0, pl.ds(j * sm, sm), :] = x_seg(c0_ccw, j, 1)
            rd = make(sb_ccw, rb_ccw, ss_ccw, rs_ccw, 0, j, left)
            rd.start()
            rdmas[("ccw", 0, j)] = rd

        for s in range(N_DEV - 1):
            c_cw = (my + 2 * N_DEV - s - 2) % N_DEV
            c_ccw = (my + s + 2) % N_DEV
            last = s == N_DEV - 2
            for j in range(SEG):
                rdmas[("cw", s, j)].wait_recv()
                acc = rb_cw[s, pl.ds(j * sm, sm), :] + x_seg(c_cw, j, 0)
                if last:
                    out_ref[pl.ds(j * sm, sm), 0:h] = acc
                else:
                    sb_cw[s + 1, pl.ds(j * sm, sm), :] = acc
                    rd = make(sb_cw, rb_cw, ss_cw, rs_cw, s + 1, j, right)
                    rd.start()
                    rdmas[("cw", s + 1, j)] = rd

                rdmas[("ccw", s, j)].wait_recv()
                acc = rb_ccw[s, pl.ds(j * sm, sm), :] + x_seg(c_ccw, j, 1)
                if last:
                    out_ref[pl.ds(j * sm, sm), h:n_per] = acc
                else:
                    sb_ccw[s + 1, pl.ds(j * sm, sm), :] = acc
                    rd = make(sb_ccw, rb_ccw, ss_ccw, rs_ccw, s + 1, j, left)
                    rd.start()
                    rdmas[("ccw", s + 1, j)] = rd

        for rd in rdmas.values():
            rd.wait_send()

    return pl.pallas_call(
        body,
        out_shape=jax.ShapeDtypeStruct((m, n_per), jnp.float32),
        in_specs=[pl.BlockSpec(memory_space=pltpu.VMEM)],
        out_specs=pl.BlockSpec(memory_space=pltpu.VMEM),
        scratch_shapes=[
            pltpu.VMEM((N_DEV - 1, m, h), jnp.float32),
            pltpu.VMEM((N_DEV - 1, m, h), jnp.float32),
            pltpu.VMEM((N_DEV - 1, m, h), jnp.float32),
            pltpu.VMEM((N_DEV - 1, m, h), jnp.float32),
            pltpu.SemaphoreType.DMA((N_DEV - 1, SEG)),
            pltpu.SemaphoreType.DMA((N_DEV - 1, SEG)),
            pltpu.SemaphoreType.DMA((N_DEV - 1, SEG)),
            pltpu.SemaphoreType.DMA((N_DEV - 1, SEG)),
        ],
        compiler_params=pltpu.CompilerParams(collective_id=0),
    )(x)


# device time: 25304 ns/iter; 1.7954x vs baseline; 1.0008x over previous
import jax
import jax.numpy as jnp
from jax import lax
from jax.experimental import pallas as pl
from jax.experimental.pallas import tpu as pltpu

N_DEV = 4
SEG = 4


def kernel(x):
    _, m, n_total = x.shape
    n_per = n_total // N_DEV
    h = n_per // 2
    sm = m // SEG

    def body(x_ref, out_ref, sb_cw, rb_cw, sb_ccw, rb_ccw,
             ss_cw, rs_cw, ss_ccw, rs_ccw):
        my = lax.axis_index("i")
        left = (my + N_DEV - 1) % N_DEV
        right = (my + 1) % N_DEV

        barrier_sem = pltpu.get_barrier_semaphore()
        for nbr in (left, right):
            pl.semaphore_signal(
                barrier_sem, inc=1,
                device_id=(nbr,), device_id_type=pl.DeviceIdType.MESH,
            )
        pl.semaphore_wait(barrier_sem, 2)

        def x_seg(c, j, half):
            return x_ref[0, pl.ds(j * sm, sm), pl.ds(c * n_per + half * h, h)]

        def make(sb, rb, ss, rs, s, j, dev):
            return pltpu.make_async_remote_copy(
                src_ref=sb.at[s, pl.ds(j * sm, sm), :],
                dst_ref=rb.at[s, pl.ds(j * sm, sm), :],
                send_sem=ss.at[s, j], recv_sem=rs.at[s, j],
                device_id=(dev,), device_id_type=pl.DeviceIdType.MESH,
            )

        rdmas = {}

        c0_cw = (my + N_DEV - 1) % N_DEV
        c0_ccw = (my + 1) % N_DEV
        for j in range(SEG):
            rd = pltpu.make_async_remote_copy(
                src_ref=x_ref.at[0, pl.ds(j * sm, sm),
                                 pl.ds(c0_cw * n_per, h)],
                dst_ref=rb_cw.at[0, pl.ds(j * sm, sm), :],
                send_sem=ss_cw.at[0, j], recv_sem=rs_cw.at[0, j],
                device_id=(right,), device_id_type=pl.DeviceIdType.MESH,
            )
            rd.start()
            rdmas[("cw", 0, j)] = rd
            rd = pltpu.make_async_remote_copy(
                src_ref=x_ref.at[0, pl.ds(j * sm, sm),
                                 pl.ds(c0_ccw * n_per + h, h)],
                dst_ref=rb_ccw.at[0, pl.ds(j * sm, sm), :],
                send_sem=ss_ccw.at[0, j], recv_sem=rs_ccw.at[0, j],
                device_id=(left,), device_id_type=pl.DeviceIdType.MESH,
            )
            rd.start()
            rdmas[("ccw", 0, j)] = rd

        for s in range(N_DEV - 1):
            c_cw = (my + 2 * N_DEV - s - 2) % N_DEV
            c_ccw = (my + s + 2) % N_DEV
            last = s == N_DEV - 2
            for j in range(SEG):
                rdmas[("cw", s, j)].wait_recv()
                acc = rb_cw[s, pl.ds(j * sm, sm), :] + x_seg(c_cw, j, 0)
                if last:
                    out_ref[pl.ds(j * sm, sm), 0:h] = acc
                else:
                    sb_cw[s + 1, pl.ds(j * sm, sm), :] = acc
                    rd = make(sb_cw, rb_cw, ss_cw, rs_cw, s + 1, j, right)
                    rd.start()
                    rdmas[("cw", s + 1, j)] = rd

                rdmas[("ccw", s, j)].wait_recv()
                acc = rb_ccw[s, pl.ds(j * sm, sm), :] + x_seg(c_ccw, j, 1)
                if last:
                    out_ref[pl.ds(j * sm, sm), h:n_per] = acc
                else:
                    sb_ccw[s + 1, pl.ds(j * sm, sm), :] = acc
                    rd = make(sb_ccw, rb_ccw, ss_ccw, rs_ccw, s + 1, j, left)
                    rd.start()
                    rdmas[("ccw", s + 1, j)] = rd

        for rd in rdmas.values():
            rd.wait_send()

    return pl.pallas_call(
        body,
        out_shape=jax.ShapeDtypeStruct((m, n_per), jnp.float32),
        in_specs=[pl.BlockSpec(memory_space=pltpu.VMEM)],
        out_specs=pl.BlockSpec(memory_space=pltpu.VMEM),
        scratch_shapes=[
            pltpu.VMEM((N_DEV - 1, m, h), jnp.float32),
            pltpu.VMEM((N_DEV - 1, m, h), jnp.float32),
            pltpu.VMEM((N_DEV - 1, m, h), jnp.float32),
            pltpu.VMEM((N_DEV - 1, m, h), jnp.float32),
            pltpu.SemaphoreType.DMA((N_DEV - 1, SEG)),
            pltpu.SemaphoreType.DMA((N_DEV - 1, SEG)),
            pltpu.SemaphoreType.DMA((N_DEV - 1, SEG)),
            pltpu.SemaphoreType.DMA((N_DEV - 1, SEG)),
        ],
        compiler_params=pltpu.CompilerParams(collective_id=0),
    )(x)


# device time: 25099 ns/iter; 1.8101x vs baseline; 1.0082x over previous
import jax
import jax.numpy as jnp
from jax import lax
from jax.experimental import pallas as pl
from jax.experimental.pallas import tpu as pltpu

N_DEV = 4
SEG = 2


def kernel(x):
    _, m, n_total = x.shape
    n_per = n_total // N_DEV
    h = n_per // 2
    sm = m // SEG

    def body(x_ref, out_ref, sb_cw, rb_cw, sb_ccw, rb_ccw,
             ss_cw, rs_cw, ss_ccw, rs_ccw):
        my = lax.axis_index("i")
        left = (my + N_DEV - 1) % N_DEV
        right = (my + 1) % N_DEV

        barrier_sem = pltpu.get_barrier_semaphore()
        for nbr in (left, right):
            pl.semaphore_signal(
                barrier_sem, inc=1,
                device_id=(nbr,), device_id_type=pl.DeviceIdType.MESH,
            )
        pl.semaphore_wait(barrier_sem, 2)

        def x_seg(c, j, half):
            return x_ref[0, pl.ds(j * sm, sm), pl.ds(c * n_per + half * h, h)]

        def make(sb, rb, ss, rs, s, j, dev):
            return pltpu.make_async_remote_copy(
                src_ref=sb.at[s, pl.ds(j * sm, sm), :],
                dst_ref=rb.at[s, pl.ds(j * sm, sm), :],
                send_sem=ss.at[s, j], recv_sem=rs.at[s, j],
                device_id=(dev,), device_id_type=pl.DeviceIdType.MESH,
            )

        rdmas = {}

        c0_cw = (my + N_DEV - 1) % N_DEV
        c0_ccw = (my + 1) % N_DEV
        for j in range(SEG):
            rd = pltpu.make_async_remote_copy(
                src_ref=x_ref.at[0, pl.ds(j * sm, sm),
                                 pl.ds(c0_cw * n_per, h)],
                dst_ref=rb_cw.at[0, pl.ds(j * sm, sm), :],
                send_sem=ss_cw.at[0, j], recv_sem=rs_cw.at[0, j],
                device_id=(right,), device_id_type=pl.DeviceIdType.MESH,
            )
            rd.start()
            rdmas[("cw", 0, j)] = rd
            rd = pltpu.make_async_remote_copy(
                src_ref=x_ref.at[0, pl.ds(j * sm, sm),
                                 pl.ds(c0_ccw * n_per + h, h)],
                dst_ref=rb_ccw.at[0, pl.ds(j * sm, sm), :],
                send_sem=ss_ccw.at[0, j], recv_sem=rs_ccw.at[0, j],
                device_id=(left,), device_id_type=pl.DeviceIdType.MESH,
            )
            rd.start()
            rdmas[("ccw", 0, j)] = rd

        for s in range(N_DEV - 1):
            c_cw = (my + 2 * N_DEV - s - 2) % N_DEV
            c_ccw = (my + s + 2) % N_DEV
            last = s == N_DEV - 2
            for j in range(SEG):
                rdmas[("cw", s, j)].wait_recv()
                acc = rb_cw[s, pl.ds(j * sm, sm), :] + x_seg(c_cw, j, 0)
                if last:
                    out_ref[pl.ds(j * sm, sm), 0:h] = acc
                else:
                    sb_cw[s + 1, pl.ds(j * sm, sm), :] = acc
                    rd = make(sb_cw, rb_cw, ss_cw, rs_cw, s + 1, j, right)
                    rd.start()
                    rdmas[("cw", s + 1, j)] = rd

                rdmas[("ccw", s, j)].wait_recv()
                acc = rb_ccw[s, pl.ds(j * sm, sm), :] + x_seg(c_ccw, j, 1)
                if last:
                    out_ref[pl.ds(j * sm, sm), h:n_per] = acc
                else:
                    sb_ccw[s + 1, pl.ds(j * sm, sm), :] = acc
                    rd = make(sb_ccw, rb_ccw, ss_ccw, rs_ccw, s + 1, j, left)
                    rd.start()
                    rdmas[("ccw", s + 1, j)] = rd

        for rd in rdmas.values():
            rd.wait_send()

    return pl.pallas_call(
        body,
        out_shape=jax.ShapeDtypeStruct((m, n_per), jnp.float32),
        in_specs=[pl.BlockSpec(memory_space=pltpu.VMEM)],
        out_specs=pl.BlockSpec(memory_space=pltpu.VMEM),
        scratch_shapes=[
            pltpu.VMEM((N_DEV - 1, m, h), jnp.float32),
            pltpu.VMEM((N_DEV - 1, m, h), jnp.float32),
            pltpu.VMEM((N_DEV - 1, m, h), jnp.float32),
            pltpu.VMEM((N_DEV - 1, m, h), jnp.float32),
            pltpu.SemaphoreType.DMA((N_DEV - 1, SEG)),
            pltpu.SemaphoreType.DMA((N_DEV - 1, SEG)),
            pltpu.SemaphoreType.DMA((N_DEV - 1, SEG)),
            pltpu.SemaphoreType.DMA((N_DEV - 1, SEG)),
        ],
        compiler_params=pltpu.CompilerParams(collective_id=0),
    )(x)
